# Initial kernel scaffold; baseline (speedup 1.0000x reference)
import jax
import jax.numpy as jnp
from jax.experimental import pallas as pl


def kernel(node_features, edge_features, edge_sh, edge_index, W_fc1, b_fc1, W_fc2, b_fc2, bn_gamma, bn_beta, eu_lin_W, eu_W1, eu_b1, eu_W2, eu_b2, eu_W3, eu_b3, ln_gamma, ln_beta):
    raise NotImplementedError("write your pallas kernel here")


# speedup vs baseline: 3.2470x; 3.2470x over previous
import functools

import numpy as np
import jax
import jax.numpy as jnp
from jax import lax
from jax.experimental import pallas as pl
from jax.experimental.pallas import tpu as pltpu
from jax.experimental.pallas import tpu_sc as plsc

N = 20000
E = 160000
NM = 16
H = 64
WN = NM * NM
ALPHA = 0.25
PAD = 32

NC, NS = 2, 16
NW = NC * NS
EPW = E // NW
CHUNK = 125
NCH = EPW // CHUNK
KFIRE = 8
NGRP = NCH // KFIRE
NPT = N // NS

_mesh = plsc.VectorSubcoreMesh(
    core_axis_name="c", subcore_axis_name="s", num_cores=NC, num_subcores=NS)

_R = np.zeros((NM, WN), np.float32)
_S = np.zeros((WN, NM), np.float32)
for _i in range(NM):
    for _k in range(NM):
        _R[_i, _i * NM + _k] = 1.0
        _S[_i * NM + _k, _k] = 1.0


@functools.partial(
    pl.kernel,
    out_type=jax.ShapeDtypeStruct((E, NM), jnp.float32),
    mesh=_mesh,
    compiler_params=pltpu.CompilerParams(use_tc_tiling_on_sc=False),
    scratch_types=[
        pltpu.VMEM((NCH, CHUNK), jnp.int32),
        pltpu.VMEM((EPW, NM), jnp.float32),
        pltpu.SemaphoreType.DMA,
    ],
)
def _gather_x(table, idx2, out, idx_v, rows_v, sem):
    w = lax.axis_index("s") * NC + lax.axis_index("c")
    pltpu.sync_copy(idx2.at[pl.ds(w * NCH, NCH), :], idx_v)

    def group(g, carry):
        base = g * KFIRE
        cps = [
            pltpu.async_copy(
                table.at[idx_v.at[base + b]],
                rows_v.at[pl.ds((base + b) * CHUNK, CHUNK), :],
                sem,
            )
            for b in range(KFIRE)
        ]
        for cp in cps:
            cp.wait()
        return carry

    lax.fori_loop(0, NGRP, group, 0)
    pltpu.sync_copy(rows_v, out.at[pl.ds(w * EPW, EPW), :])


def _edge_tp_body(ef, x, sh, w1, b1, w2, b2, r, s, out):
    h = jnp.maximum(ef[...] @ w1[...] + b1[...], 0.0)
    wmat = h @ w2[...] + b2[...]
    xr = (x[...] * (ALPHA * sh[...])) @ r[...]
    tp = (xr * wmat) @ s[...]
    blk = tp.shape[0]
    out[...] = jnp.concatenate(
        [tp, jnp.ones((blk, 1), jnp.float32), jnp.zeros((blk, PAD - NM - 1), jnp.float32)],
        axis=1)


_BE = 2000

_edge_tp = pl.pallas_call(
    _edge_tp_body,
    grid=(E // _BE,),
    in_specs=[
        pl.BlockSpec((_BE, H), lambda i: (i, 0)),
        pl.BlockSpec((_BE, NM), lambda i: (i, 0)),
        pl.BlockSpec((_BE, 1), lambda i: (i, 0)),
        pl.BlockSpec((H, H), lambda i: (0, 0)),
        pl.BlockSpec((1, H), lambda i: (0, 0)),
        pl.BlockSpec((H, WN), lambda i: (0, 0)),
        pl.BlockSpec((1, WN), lambda i: (0, 0)),
        pl.BlockSpec((NM, WN), lambda i: (0, 0)),
        pl.BlockSpec((WN, NM), lambda i: (0, 0)),
    ],
    out_specs=pl.BlockSpec((_BE, PAD), lambda i: (i, 0)),
    out_shape=jax.ShapeDtypeStruct((E, PAD), jnp.float32),
    compiler_params=pltpu.CompilerParams(dimension_semantics=("parallel",)),
)


@functools.partial(
    pl.kernel,
    out_type=jax.ShapeDtypeStruct((NC, N, PAD), jnp.float32),
    mesh=_mesh,
    compiler_params=pltpu.CompilerParams(use_tc_tiling_on_sc=False),
    scratch_types=[
        pltpu.VMEM((NCH, CHUNK), jnp.int32),
        pltpu.VMEM((KFIRE, CHUNK, PAD), jnp.float32),
        pltpu.VMEM_SHARED((N, PAD), jnp.float32),
        pltpu.SemaphoreType.DMA,
    ],
)
def _scatter_tp(tp3, src2, zeros_hbm, out, idx_v, rows_v, acc, sem):
    c = lax.axis_index("c")
    s = lax.axis_index("s")
    w = s * NC + c
    pltpu.sync_copy(zeros_hbm.at[pl.ds(s * NPT, NPT), :],
                    acc.at[pl.ds(s * NPT, NPT), :])
    pltpu.sync_copy(src2.at[pl.ds(w * NCH, NCH), :], idx_v)
    plsc.subcore_barrier()

    def group(g, carry):
        base = g * KFIRE
        cps = [
            pltpu.async_copy(tp3.at[w * NCH + base + b], rows_v.at[b], sem)
            for b in range(KFIRE)
        ]
        for cp in cps:
            cp.wait()
        for b in range(KFIRE):
            pltpu.sync_copy(rows_v.at[b], acc.at[idx_v.at[base + b]], add=True)
        return carry

    lax.fori_loop(0, NGRP, group, 0)
    plsc.subcore_barrier()
    pltpu.sync_copy(acc.at[pl.ds(s * NPT, NPT), :],
                    out.at[c, pl.ds(s * NPT, NPT), :])


def _combine_bn_body(p0, p1, nf, gam, bet, out):
    t = p0[...] + p1[...]
    sums = t[:, :NM]
    cnt = t[:, NM:NM + 1]
    o = sums / jnp.maximum(cnt, 1.0) + nf[...]
    mu = jnp.mean(o, axis=0, keepdims=True)
    var = jnp.mean((o - mu) ** 2, axis=0, keepdims=True)
    out[...] = (o - mu) * lax.rsqrt(var + 1e-5) * gam[...] + bet[...]


_combine_bn = pl.pallas_call(
    _combine_bn_body,
    out_shape=jax.ShapeDtypeStruct((N, NM), jnp.float32),
)


@functools.partial(
    pl.kernel,
    out_type=(jax.ShapeDtypeStruct((E, NM), jnp.float32),
              jax.ShapeDtypeStruct((E, NM), jnp.float32)),
    mesh=_mesh,
    compiler_params=pltpu.CompilerParams(use_tc_tiling_on_sc=False),
    scratch_types=[
        pltpu.VMEM((NCH, CHUNK), jnp.int32),
        pltpu.VMEM((NCH, CHUNK), jnp.int32),
        pltpu.VMEM((EPW, NM), jnp.float32),
        pltpu.SemaphoreType.DMA,
    ],
)
def _gather_out(table, dst2, src2, o_dst, o_src, idx_d, idx_s, rows_v, sem):
    w = lax.axis_index("s") * NC + lax.axis_index("c")
    pltpu.sync_copy(dst2.at[pl.ds(w * NCH, NCH), :], idx_d)
    pltpu.sync_copy(src2.at[pl.ds(w * NCH, NCH), :], idx_s)

    def run(idx_v, dst_hbm):
        def group(g, carry):
            base = g * KFIRE
            cps = [
                pltpu.async_copy(
                    table.at[idx_v.at[base + b]],
                    rows_v.at[pl.ds((base + b) * CHUNK, CHUNK), :],
                    sem,
                )
                for b in range(KFIRE)
            ]
            for cp in cps:
                cp.wait()
            return carry

        lax.fori_loop(0, NGRP, group, 0)
        pltpu.sync_copy(rows_v, dst_hbm.at[pl.ds(w * EPW, EPW), :])

    run(idx_d, o_dst)
    run(idx_s, o_src)


def _edge_update_body(ef, od, os_, lin, w1, b1, w2, b2, w3, b3, gam, bet, out):
    lin4 = lin[...] * 0.25
    ud = lin4 @ w1[:H, :]
    us = lin4 @ w1[H:2 * H, :]
    m = od[...] @ ud + os_[...] @ us + ef[...] @ w1[2 * H:, :] + b1[...]
    m = jnp.maximum(m, 0.0)
    m = jnp.maximum(m @ w2[...] + b2[...], 0.0)
    m = m @ w3[...] + b3[...]
    ef2 = ef[...] + m
    mu = jnp.mean(ef2, axis=1, keepdims=True)
    var = jnp.mean((ef2 - mu) ** 2, axis=1, keepdims=True)
    out[...] = (ef2 - mu) * lax.rsqrt(var + 1e-5) * gam[...] + bet[...]


_edge_update = pl.pallas_call(
    _edge_update_body,
    grid=(E // _BE,),
    in_specs=[
        pl.BlockSpec((_BE, H), lambda i: (i, 0)),
        pl.BlockSpec((_BE, NM), lambda i: (i, 0)),
        pl.BlockSpec((_BE, NM), lambda i: (i, 0)),
        pl.BlockSpec((NM, H), lambda i: (0, 0)),
        pl.BlockSpec((3 * H, H), lambda i: (0, 0)),
        pl.BlockSpec((1, H), lambda i: (0, 0)),
        pl.BlockSpec((H, H), lambda i: (0, 0)),
        pl.BlockSpec((1, H), lambda i: (0, 0)),
        pl.BlockSpec((H, H), lambda i: (0, 0)),
        pl.BlockSpec((1, H), lambda i: (0, 0)),
        pl.BlockSpec((1, H), lambda i: (0, 0)),
        pl.BlockSpec((1, H), lambda i: (0, 0)),
    ],
    out_specs=pl.BlockSpec((_BE, H), lambda i: (i, 0)),
    out_shape=jax.ShapeDtypeStruct((E, H), jnp.float32),
    compiler_params=pltpu.CompilerParams(dimension_semantics=("parallel",)),
)


def kernel(node_features, edge_features, edge_sh, edge_index,
           W_fc1, b_fc1, W_fc2, b_fc2, bn_gamma, bn_beta,
           eu_lin_W, eu_W1, eu_b1, eu_W2, eu_b2, eu_W3, eu_b3,
           ln_gamma, ln_beta):
    dst2 = edge_index[0].reshape(E // CHUNK, CHUNK)
    src2 = edge_index[1].reshape(E // CHUNK, CHUNK)

    x = _gather_x(node_features, dst2)
    tp_pad = _edge_tp(edge_features, x, edge_sh,
                      W_fc1, b_fc1.reshape(1, H), W_fc2, b_fc2.reshape(1, WN),
                      _R, _S)
    partials = _scatter_tp(tp_pad.reshape(E // CHUNK, CHUNK, PAD), src2,
                           jnp.zeros((N, PAD), jnp.float32))
    out = _combine_bn(partials[0], partials[1], node_features,
                      bn_gamma.reshape(1, NM), bn_beta.reshape(1, NM))
    o_dst, o_src = _gather_out(out, dst2, src2)
    ef = _edge_update(edge_features, o_dst, o_src, eu_lin_W, eu_W1,
                      eu_b1.reshape(1, H), eu_W2, eu_b2.reshape(1, H),
                      eu_W3, eu_b3.reshape(1, H),
                      ln_gamma.reshape(1, H), ln_beta.reshape(1, H))
    return (out, ef)


# speedup vs baseline: 4.8341x; 1.4888x over previous
import functools

import numpy as np
import jax
import jax.numpy as jnp
from jax import lax
from jax.experimental import pallas as pl
from jax.experimental.pallas import tpu as pltpu
from jax.experimental.pallas import tpu_sc as plsc

N = 20000
E = 160000
NM = 16
H = 64
WN = NM * NM
ALPHA = 0.25
PAD = 32

NC, NS = 2, 16
NW = NC * NS
CHUNK = 125
NCH = 40
KFIRE = 8
NGRP = NCH // KFIRE
EPW = NCH * CHUNK
NPT = N // NS

_mesh = plsc.VectorSubcoreMesh(
    core_axis_name="c", subcore_axis_name="s", num_cores=NC, num_subcores=NS)
_sc_params = pltpu.CompilerParams(use_tc_tiling_on_sc=False)


@functools.partial(
    pl.kernel,
    out_type=jax.ShapeDtypeStruct((E, NM), jnp.float32),
    mesh=_mesh,
    compiler_params=_sc_params,
    scratch_types=[
        pltpu.VMEM((NCH, CHUNK), jnp.int32),
        pltpu.VMEM((EPW, NM), jnp.float32),
        pltpu.SemaphoreType.DMA,
    ],
)
def _gather_x(table, idx2, out, idx_v, rows_v, sem):
    w = lax.axis_index("s") * NC + lax.axis_index("c")
    pltpu.sync_copy(idx2.at[pl.ds(w * NCH, NCH), :], idx_v)

    def group(g, carry):
        base = g * KFIRE
        cps = [
            pltpu.async_copy(
                table.at[idx_v.at[base + b]],
                rows_v.at[pl.ds((base + b) * CHUNK, CHUNK), :],
                sem,
            )
            for b in range(KFIRE)
        ]
        for cp in cps:
            cp.wait()
        return carry

    lax.fori_loop(0, NGRP, group, 0)
    pltpu.sync_copy(rows_v, out.at[pl.ds(w * EPW, EPW), :])


_BT = 1280


def _unpack16(packed, eye):
    parts = [
        lax.dot_general(eye, packed[:, 16 * j:16 * (j + 1)],
                        (((1,), (1,)), ((), ())))
        for j in range(8)
    ]
    return jnp.concatenate(parts, axis=1)


def _edge_tp_body(eft, xp, sht, w1t, b1c, w2t, b2c, out):
    eye = jnp.eye(NM, dtype=jnp.float32)
    ht = jnp.maximum(w1t[...] @ eft[...] + b1c[...], 0.0)
    wt = w2t[...] @ ht + b2c[...]
    xt = _unpack16(xp[...], eye)
    xsht = xt * (ALPHA * sht[...])
    tpt = xsht[0:1, :] * wt[0:NM, :]
    for i in range(1, NM):
        tpt = tpt + xsht[i:i + 1, :] * wt[i * NM:(i + 1) * NM, :]
    rows = lax.dot_general(tpt, eye, (((0,), (0,)), ((), ())))
    out[...] = jnp.concatenate(
        [rows, jnp.ones((_BT, 1), jnp.float32),
         jnp.zeros((_BT, PAD - NM - 1), jnp.float32)], axis=1)


_edge_tp = pl.pallas_call(
    _edge_tp_body,
    grid=(E // _BT,),
    in_specs=[
        pl.BlockSpec((H, _BT), lambda i: (0, i)),
        pl.BlockSpec((_BT // 8, 128), lambda i: (i, 0)),
        pl.BlockSpec((1, _BT), lambda i: (0, i)),
        pl.BlockSpec((H, H), lambda i: (0, 0)),
        pl.BlockSpec((H, 1), lambda i: (0, 0)),
        pl.BlockSpec((WN, H), lambda i: (0, 0)),
        pl.BlockSpec((WN, 1), lambda i: (0, 0)),
    ],
    out_specs=pl.BlockSpec((_BT, PAD), lambda i: (i, 0)),
    out_shape=jax.ShapeDtypeStruct((E, PAD), jnp.float32),
    compiler_params=pltpu.CompilerParams(dimension_semantics=("parallel",)),
)


@functools.partial(
    pl.kernel,
    out_type=jax.ShapeDtypeStruct((NC, N, PAD), jnp.float32),
    mesh=_mesh,
    compiler_params=_sc_params,
    scratch_types=[
        pltpu.VMEM((NCH, CHUNK), jnp.int32),
        pltpu.VMEM((KFIRE, CHUNK, PAD), jnp.float32),
        pltpu.VMEM_SHARED((N, PAD), jnp.float32),
        pltpu.SemaphoreType.DMA,
    ],
)
def _scatter_tp(tp2, src2, zeros_hbm, out, idx_v, rows_v, acc, sem):
    c = lax.axis_index("c")
    s = lax.axis_index("s")
    w = s * NC + c
    pltpu.sync_copy(zeros_hbm.at[pl.ds(s * NPT, NPT), :],
                    acc.at[pl.ds(s * NPT, NPT), :])
    pltpu.sync_copy(src2.at[pl.ds(w * NCH, NCH), :], idx_v)
    plsc.subcore_barrier()

    def group(g, carry):
        base = g * KFIRE
        cps = [
            pltpu.async_copy(
                tp2.at[pl.ds((w * NCH + base + b) * CHUNK, CHUNK), :],
                rows_v.at[b], sem)
            for b in range(KFIRE)
        ]
        for cp in cps:
            cp.wait()
        for b in range(KFIRE):
            pltpu.sync_copy(rows_v.at[b], acc.at[idx_v.at[base + b]], add=True)
        return carry

    lax.fori_loop(0, NGRP, group, 0)
    plsc.subcore_barrier()
    pltpu.sync_copy(acc.at[pl.ds(s * NPT, NPT), :],
                    out.at[c, pl.ds(s * NPT, NPT), :])


def _combine_bn_body(p, nf, gamc, betc, out_sc, out_t):
    t = p[0] + p[1]
    sums = t[:, :NM]
    cnt = t[:, NM:NM + 1]
    o = sums / jnp.maximum(cnt, 1.0) + nf[...]
    mu = jnp.mean(o, axis=0, keepdims=True)
    var = jnp.mean((o - mu) ** 2, axis=0, keepdims=True)
    o = (o - mu) * lax.rsqrt(var + 1e-5) * gamc[...].T + betc[...].T
    out_sc[...] = o
    out_t[...] = lax.dot_general(jnp.eye(NM, dtype=jnp.float32), o,
                                 (((1,), (1,)), ((), ())))


_combine_bn = pl.pallas_call(
    _combine_bn_body,
    out_shape=(jax.ShapeDtypeStruct((N, NM), jnp.float32),
               jax.ShapeDtypeStruct((NM, N), jnp.float32)),
)


@functools.partial(
    pl.kernel,
    out_type=(jax.ShapeDtypeStruct((E, NM), jnp.float32),
              jax.ShapeDtypeStruct((E, NM), jnp.float32)),
    mesh=_mesh,
    compiler_params=_sc_params,
    scratch_types=[
        pltpu.VMEM((NCH, CHUNK), jnp.int32),
        pltpu.VMEM((NCH, CHUNK), jnp.int32),
        pltpu.VMEM((EPW, NM), jnp.float32),
        pltpu.SemaphoreType.DMA,
    ],
)
def _gather_out(table, dst2, src2, o_dst, o_src, idx_d, idx_s, rows_v, sem):
    w = lax.axis_index("s") * NC + lax.axis_index("c")
    pltpu.sync_copy(dst2.at[pl.ds(w * NCH, NCH), :], idx_d)
    pltpu.sync_copy(src2.at[pl.ds(w * NCH, NCH), :], idx_s)

    def run(idx_v, dst_hbm):
        def group(g, carry):
            base = g * KFIRE
            cps = [
                pltpu.async_copy(
                    table.at[idx_v.at[base + b]],
                    rows_v.at[pl.ds((base + b) * CHUNK, CHUNK), :],
                    sem,
                )
                for b in range(KFIRE)
            ]
            for cp in cps:
                cp.wait()
            return carry

        lax.fori_loop(0, NGRP, group, 0)
        pltpu.sync_copy(rows_v, dst_hbm.at[pl.ds(w * EPW, EPW), :])

    run(idx_d, o_dst)
    run(idx_s, o_src)


def _edge_update_body(eft, od_p, os_p, lin, w1, b1c, w2t, b2c, w3t, b3c,
                      lngc, lnbc, out):
    eye = jnp.eye(NM, dtype=jnp.float32)
    odt = _unpack16(od_p[...], eye)
    ost = _unpack16(os_p[...], eye)
    lin4 = lin[...] * 0.25
    ad = (lin4 @ w1[0:H, :]).T
    as_ = (lin4 @ w1[H:2 * H, :]).T
    mt = (ad @ odt + as_ @ ost + w1[2 * H:, :].T @ eft[...]
          + b1c[...])
    mt = jnp.maximum(mt, 0.0)
    mt = jnp.maximum(w2t[...] @ mt + b2c[...], 0.0)
    mt = w3t[...] @ mt + b3c[...]
    ef2 = eft[...] + mt
    mu = jnp.mean(ef2, axis=0, keepdims=True)
    var = jnp.mean((ef2 - mu) ** 2, axis=0, keepdims=True)
    out[...] = (ef2 - mu) * lax.rsqrt(var + 1e-5) * lngc[...] + lnbc[...]


_edge_update = pl.pallas_call(
    _edge_update_body,
    grid=(E // _BT,),
    in_specs=[
        pl.BlockSpec((H, _BT), lambda i: (0, i)),
        pl.BlockSpec((_BT // 8, 128), lambda i: (i, 0)),
        pl.BlockSpec((_BT // 8, 128), lambda i: (i, 0)),
        pl.BlockSpec((NM, H), lambda i: (0, 0)),
        pl.BlockSpec((3 * H, H), lambda i: (0, 0)),
        pl.BlockSpec((H, 1), lambda i: (0, 0)),
        pl.BlockSpec((H, H), lambda i: (0, 0)),
        pl.BlockSpec((H, 1), lambda i: (0, 0)),
        pl.BlockSpec((H, H), lambda i: (0, 0)),
        pl.BlockSpec((H, 1), lambda i: (0, 0)),
        pl.BlockSpec((H, 1), lambda i: (0, 0)),
        pl.BlockSpec((H, 1), lambda i: (0, 0)),
    ],
    out_specs=pl.BlockSpec((H, _BT), lambda i: (0, i)),
    out_shape=jax.ShapeDtypeStruct((H, E), jnp.float32),
    compiler_params=pltpu.CompilerParams(dimension_semantics=("parallel",)),
)


def kernel(node_features, edge_features, edge_sh, edge_index,
           W_fc1, b_fc1, W_fc2, b_fc2, bn_gamma, bn_beta,
           eu_lin_W, eu_W1, eu_b1, eu_W2, eu_b2, eu_W3, eu_b3,
           ln_gamma, ln_beta):
    src2 = edge_index[1].reshape(E // CHUNK, CHUNK)
    dstp = (edge_index[0].reshape(E // _BT, 8, _BT // 8)
            .swapaxes(1, 2).reshape(E // CHUNK, CHUNK))
    srcp = (edge_index[1].reshape(E // _BT, 8, _BT // 8)
            .swapaxes(1, 2).reshape(E // CHUNK, CHUNK))
    eft = edge_features.T
    sht = edge_sh.T

    x = _gather_x(node_features, dstp)
    tp = _edge_tp(eft, x.reshape(E // 8, 128), sht,
                  W_fc1.T, b_fc1.reshape(H, 1),
                  W_fc2.T, b_fc2.reshape(WN, 1))
    partials = _scatter_tp(tp, src2,
                           jnp.zeros((N, PAD), jnp.float32))
    out_sc, out_t = _combine_bn(partials, node_features,
                                bn_gamma.reshape(NM, 1),
                                bn_beta.reshape(NM, 1))
    o_dst, o_src = _gather_out(out_sc, dstp, srcp)
    eft_out = _edge_update(eft, o_dst.reshape(E // 8, 128),
                           o_src.reshape(E // 8, 128),
                           eu_lin_W, eu_W1,
                           eu_b1.reshape(H, 1), eu_W2.T,
                           eu_b2.reshape(H, 1), eu_W3.T,
                           eu_b3.reshape(H, 1),
                           ln_gamma.reshape(H, 1),
                           ln_beta.reshape(H, 1))
    return (out_t.T, eft_out.T)


# speedup vs baseline: 5.1213x; 1.0594x over previous
import functools

import numpy as np
import jax
import jax.numpy as jnp
from jax import lax
from jax.experimental import pallas as pl
from jax.experimental.pallas import tpu as pltpu
from jax.experimental.pallas import tpu_sc as plsc

N = 20000
E = 160000
NM = 16
H = 64
WN = NM * NM
ALPHA = 0.25
PAD = 32

NC, NS = 2, 16
NW = NC * NS
CHUNK = 125
NCH = 40
KFIRE = 8
NGRP = NCH // KFIRE
EPW = NCH * CHUNK
NPT = N // NS

_mesh = plsc.VectorSubcoreMesh(
    core_axis_name="c", subcore_axis_name="s", num_cores=NC, num_subcores=NS)
_sc_params = pltpu.CompilerParams(use_tc_tiling_on_sc=False)


@functools.partial(
    pl.kernel,
    out_type=jax.ShapeDtypeStruct((E, NM), jnp.float32),
    mesh=_mesh,
    compiler_params=_sc_params,
    scratch_types=[
        pltpu.VMEM((NCH, CHUNK), jnp.int32),
        pltpu.VMEM((EPW, NM), jnp.float32),
        pltpu.SemaphoreType.DMA,
    ],
)
def _gather_x(table, idx2, out, idx_v, rows_v, sem):
    w = lax.axis_index("s") * NC + lax.axis_index("c")
    pltpu.sync_copy(idx2.at[pl.ds(w * NCH, NCH), :], idx_v)

    def group(g, carry):
        base = g * KFIRE
        cps = [
            pltpu.async_copy(
                table.at[idx_v.at[base + b]],
                rows_v.at[pl.ds((base + b) * CHUNK, CHUNK), :],
                sem,
            )
            for b in range(KFIRE)
        ]
        for cp in cps:
            cp.wait()
        return carry

    lax.fori_loop(0, NGRP, group, 0)
    pltpu.sync_copy(rows_v, out.at[pl.ds(w * EPW, EPW), :])


_BT = 1280


def _unpack16(packed, eye):
    parts = [
        lax.dot_general(eye, packed[:, 16 * j:16 * (j + 1)],
                        (((1,), (1,)), ((), ())))
        for j in range(8)
    ]
    return jnp.concatenate(parts, axis=1)


def _edge_tp_body(eft, xp, sht, w1t, b1c, w2t, b2c, out):
    eye = jnp.eye(NM, dtype=jnp.float32)
    ht = jnp.maximum(w1t[...] @ eft[...] + b1c[...], 0.0)
    wt = w2t[...] @ ht + b2c[...]
    xt = _unpack16(xp[...], eye)
    xsht = xt * (ALPHA * sht[...])
    tpt = xsht[0:1, :] * wt[0:NM, :]
    for i in range(1, NM):
        tpt = tpt + xsht[i:i + 1, :] * wt[i * NM:(i + 1) * NM, :]
    q = _BT // 4
    pieces = []
    for j in range(4):
        tj = lax.dot_general(tpt[:, j * q:(j + 1) * q], eye,
                             (((0,), (0,)), ((), ())))
        pieces.append(jnp.concatenate(
            [tj, jnp.ones((q, 1), jnp.float32),
             jnp.zeros((q, PAD - NM - 1), jnp.float32)], axis=1))
    out[...] = jnp.concatenate(pieces, axis=1)


_edge_tp = pl.pallas_call(
    _edge_tp_body,
    grid=(E // _BT,),
    in_specs=[
        pl.BlockSpec((H, _BT), lambda i: (0, i)),
        pl.BlockSpec((_BT // 8, 128), lambda i: (i, 0)),
        pl.BlockSpec((1, _BT), lambda i: (0, i)),
        pl.BlockSpec((H, H), lambda i: (0, 0)),
        pl.BlockSpec((H, 1), lambda i: (0, 0)),
        pl.BlockSpec((WN, H), lambda i: (0, 0)),
        pl.BlockSpec((WN, 1), lambda i: (0, 0)),
    ],
    out_specs=pl.BlockSpec((_BT // 4, 128), lambda i: (i, 0)),
    out_shape=jax.ShapeDtypeStruct((E // 4, 128), jnp.float32),
    compiler_params=pltpu.CompilerParams(dimension_semantics=("parallel",)),
)


@functools.partial(
    pl.kernel,
    out_type=jax.ShapeDtypeStruct((NC, N, PAD), jnp.float32),
    mesh=_mesh,
    compiler_params=_sc_params,
    scratch_types=[
        pltpu.VMEM((NCH, CHUNK), jnp.int32),
        pltpu.VMEM((KFIRE, CHUNK, PAD), jnp.float32),
        pltpu.VMEM_SHARED((N, PAD), jnp.float32),
        pltpu.SemaphoreType.DMA,
    ],
)
def _scatter_tp(tp2, src2, zeros_hbm, out, idx_v, rows_v, acc, sem):
    c = lax.axis_index("c")
    s = lax.axis_index("s")
    w = s * NC + c
    pltpu.sync_copy(zeros_hbm.at[pl.ds(s * NPT, NPT), :],
                    acc.at[pl.ds(s * NPT, NPT), :])
    pltpu.sync_copy(src2.at[pl.ds(w * NCH, NCH), :], idx_v)
    plsc.subcore_barrier()

    def group(g, carry):
        base = g * KFIRE
        cps = [
            pltpu.async_copy(
                tp2.at[pl.ds((w * NCH + base + b) * CHUNK, CHUNK), :],
                rows_v.at[b], sem)
            for b in range(KFIRE)
        ]
        for cp in cps:
            cp.wait()
        for b in range(KFIRE):
            pltpu.sync_copy(rows_v.at[b], acc.at[idx_v.at[base + b]], add=True)
        return carry

    lax.fori_loop(0, NGRP, group, 0)
    plsc.subcore_barrier()
    pltpu.sync_copy(acc.at[pl.ds(s * NPT, NPT), :],
                    out.at[c, pl.ds(s * NPT, NPT), :])


def _combine_bn_body(p, nf, gamc, betc, out_sc, out_t):
    t = p[0] + p[1]
    sums = t[:, :NM]
    cnt = t[:, NM:NM + 1]
    o = sums / jnp.maximum(cnt, 1.0) + nf[...]
    mu = jnp.mean(o, axis=0, keepdims=True)
    var = jnp.mean((o - mu) ** 2, axis=0, keepdims=True)
    o = (o - mu) * lax.rsqrt(var + 1e-5) * gamc[...].T + betc[...].T
    out_sc[...] = o
    out_t[...] = lax.dot_general(jnp.eye(NM, dtype=jnp.float32), o,
                                 (((1,), (1,)), ((), ())))


_combine_bn = pl.pallas_call(
    _combine_bn_body,
    out_shape=(jax.ShapeDtypeStruct((N, NM), jnp.float32),
               jax.ShapeDtypeStruct((NM, N), jnp.float32)),
)


@functools.partial(
    pl.kernel,
    out_type=(jax.ShapeDtypeStruct((E, NM), jnp.float32),
              jax.ShapeDtypeStruct((E, NM), jnp.float32)),
    mesh=_mesh,
    compiler_params=_sc_params,
    scratch_types=[
        pltpu.VMEM((NCH, CHUNK), jnp.int32),
        pltpu.VMEM((NCH, CHUNK), jnp.int32),
        pltpu.VMEM((EPW, NM), jnp.float32),
        pltpu.SemaphoreType.DMA,
    ],
)
def _gather_out(table, dst2, src2, o_dst, o_src, idx_d, idx_s, rows_v, sem):
    w = lax.axis_index("s") * NC + lax.axis_index("c")
    pltpu.sync_copy(dst2.at[pl.ds(w * NCH, NCH), :], idx_d)
    pltpu.sync_copy(src2.at[pl.ds(w * NCH, NCH), :], idx_s)

    def run(idx_v, dst_hbm):
        def group(g, carry):
            base = g * KFIRE
            cps = [
                pltpu.async_copy(
                    table.at[idx_v.at[base + b]],
                    rows_v.at[pl.ds((base + b) * CHUNK, CHUNK), :],
                    sem,
                )
                for b in range(KFIRE)
            ]
            for cp in cps:
                cp.wait()
            return carry

        lax.fori_loop(0, NGRP, group, 0)
        pltpu.sync_copy(rows_v, dst_hbm.at[pl.ds(w * EPW, EPW), :])

    run(idx_d, o_dst)
    run(idx_s, o_src)


def _edge_update_body(eft, od_p, os_p, lin, w1, b1c, w2t, b2c, w3t, b3c,
                      lngc, lnbc, out):
    eye = jnp.eye(NM, dtype=jnp.float32)
    odt = _unpack16(od_p[...], eye)
    ost = _unpack16(os_p[...], eye)
    lin4 = lin[...] * 0.25
    ad = (lin4 @ w1[0:H, :]).T
    as_ = (lin4 @ w1[H:2 * H, :]).T
    mt = (ad @ odt + as_ @ ost + w1[2 * H:, :].T @ eft[...]
          + b1c[...])
    mt = jnp.maximum(mt, 0.0)
    mt = jnp.maximum(w2t[...] @ mt + b2c[...], 0.0)
    mt = w3t[...] @ mt + b3c[...]
    ef2 = eft[...] + mt
    mu = jnp.mean(ef2, axis=0, keepdims=True)
    var = jnp.mean((ef2 - mu) ** 2, axis=0, keepdims=True)
    out[...] = (ef2 - mu) * lax.rsqrt(var + 1e-5) * lngc[...] + lnbc[...]


_edge_update = pl.pallas_call(
    _edge_update_body,
    grid=(E // _BT,),
    in_specs=[
        pl.BlockSpec((H, _BT), lambda i: (0, i)),
        pl.BlockSpec((_BT // 8, 128), lambda i: (i, 0)),
        pl.BlockSpec((_BT // 8, 128), lambda i: (i, 0)),
        pl.BlockSpec((NM, H), lambda i: (0, 0)),
        pl.BlockSpec((3 * H, H), lambda i: (0, 0)),
        pl.BlockSpec((H, 1), lambda i: (0, 0)),
        pl.BlockSpec((H, H), lambda i: (0, 0)),
        pl.BlockSpec((H, 1), lambda i: (0, 0)),
        pl.BlockSpec((H, H), lambda i: (0, 0)),
        pl.BlockSpec((H, 1), lambda i: (0, 0)),
        pl.BlockSpec((H, 1), lambda i: (0, 0)),
        pl.BlockSpec((H, 1), lambda i: (0, 0)),
    ],
    out_specs=pl.BlockSpec((H, _BT), lambda i: (0, i)),
    out_shape=jax.ShapeDtypeStruct((H, E), jnp.float32),
    compiler_params=pltpu.CompilerParams(dimension_semantics=("parallel",)),
)


def kernel(node_features, edge_features, edge_sh, edge_index,
           W_fc1, b_fc1, W_fc2, b_fc2, bn_gamma, bn_beta,
           eu_lin_W, eu_W1, eu_b1, eu_W2, eu_b2, eu_W3, eu_b3,
           ln_gamma, ln_beta):
    dstp = (edge_index[0].reshape(E // _BT, 8, _BT // 8)
            .swapaxes(1, 2).reshape(E // CHUNK, CHUNK))
    srcp = (edge_index[1].reshape(E // _BT, 8, _BT // 8)
            .swapaxes(1, 2).reshape(E // CHUNK, CHUNK))
    srcp4 = (edge_index[1].reshape(E // _BT, 4, _BT // 4)
             .swapaxes(1, 2).reshape(E // CHUNK, CHUNK))
    eft = edge_features.T
    sht = edge_sh.T

    x = _gather_x(node_features, dstp)
    tp = _edge_tp(eft, x.reshape(E // 8, 128), sht,
                  W_fc1.T, b_fc1.reshape(H, 1),
                  W_fc2.T, b_fc2.reshape(WN, 1))
    partials = _scatter_tp(tp.reshape(E, PAD), srcp4,
                           jnp.zeros((N, PAD), jnp.float32))
    out_sc, out_t = _combine_bn(partials, node_features,
                                bn_gamma.reshape(NM, 1),
                                bn_beta.reshape(NM, 1))
    o_dst, o_src = _gather_out(out_sc, dstp, srcp)
    eft_out = _edge_update(eft, o_dst.reshape(E // 8, 128),
                           o_src.reshape(E // 8, 128),
                           eu_lin_W, eu_W1,
                           eu_b1.reshape(H, 1), eu_W2.T,
                           eu_b2.reshape(H, 1), eu_W3.T,
                           eu_b3.reshape(H, 1),
                           ln_gamma.reshape(H, 1),
                           ln_beta.reshape(H, 1))
    return (out_t.T, eft_out.T)


# speedup vs baseline: 5.3917x; 1.0528x over previous
import functools

import numpy as np
import jax
import jax.numpy as jnp
from jax import lax
from jax.experimental import pallas as pl
from jax.experimental.pallas import tpu as pltpu
from jax.experimental.pallas import tpu_sc as plsc

N = 20000
E = 160000
NM = 16
H = 64
WN = NM * NM
ALPHA = 0.25
PAD = 32

NC, NS = 2, 16
NW = NC * NS
CHUNK = 125
NCH = 40
KFIRE = 8
NGRP = NCH // KFIRE
EPW = NCH * CHUNK
NPT = N // NS

_mesh = plsc.VectorSubcoreMesh(
    core_axis_name="c", subcore_axis_name="s", num_cores=NC, num_subcores=NS)
_sc_params = pltpu.CompilerParams(use_tc_tiling_on_sc=False)


@functools.partial(
    pl.kernel,
    out_type=jax.ShapeDtypeStruct((E, NM), jnp.float32),
    mesh=_mesh,
    compiler_params=_sc_params,
    scratch_types=[
        pltpu.VMEM((NCH, CHUNK), jnp.int32),
        pltpu.VMEM((EPW, NM), jnp.float32),
        pltpu.SemaphoreType.DMA,
    ],
)
def _gather_x(table, idx2, out, idx_v, rows_v, sem):
    w = lax.axis_index("s") * NC + lax.axis_index("c")
    pltpu.sync_copy(idx2.at[pl.ds(w * NCH, NCH), :], idx_v)

    def group(g, carry):
        base = g * KFIRE
        cps = [
            pltpu.async_copy(
                table.at[idx_v.at[base + b]],
                rows_v.at[pl.ds((base + b) * CHUNK, CHUNK), :],
                sem,
            )
            for b in range(KFIRE)
        ]
        for cp in cps:
            cp.wait()
        return carry

    lax.fori_loop(0, NGRP, group, 0)
    pltpu.sync_copy(rows_v, out.at[pl.ds(w * EPW, EPW), :])


_BT = 1280


def _unpack16(packed, eye):
    parts = [
        lax.dot_general(eye, packed[:, 16 * j:16 * (j + 1)],
                        (((1,), (1,)), ((), ())))
        for j in range(8)
    ]
    return jnp.concatenate(parts, axis=1)


def _edge_tp_body(eft, xp, sht, w1t, b1c, w2t, b2c, out):
    eye = jnp.eye(NM, dtype=jnp.float32)
    ht = jnp.maximum(w1t[...] @ eft[...] + b1c[...], 0.0)
    wt = w2t[...] @ ht + b2c[...]
    xt = _unpack16(xp[...], eye)
    xsht = xt * (ALPHA * sht[...])
    tpt = xsht[0:1, :] * wt[0:NM, :]
    for i in range(1, NM):
        tpt = tpt + xsht[i:i + 1, :] * wt[i * NM:(i + 1) * NM, :]
    q = _BT // 4
    pieces = []
    for j in range(4):
        tj = lax.dot_general(tpt[:, j * q:(j + 1) * q], eye,
                             (((0,), (0,)), ((), ())))
        pieces.append(jnp.concatenate(
            [tj, jnp.ones((q, 1), jnp.float32),
             jnp.zeros((q, PAD - NM - 1), jnp.float32)], axis=1))
    out[...] = jnp.concatenate(pieces, axis=1)


_edge_tp = pl.pallas_call(
    _edge_tp_body,
    grid=(E // _BT,),
    in_specs=[
        pl.BlockSpec((H, _BT), lambda i: (0, i)),
        pl.BlockSpec((_BT // 8, 128), lambda i: (i, 0)),
        pl.BlockSpec((1, _BT), lambda i: (0, i)),
        pl.BlockSpec((H, H), lambda i: (0, 0)),
        pl.BlockSpec((H, 1), lambda i: (0, 0)),
        pl.BlockSpec((WN, H), lambda i: (0, 0)),
        pl.BlockSpec((WN, 1), lambda i: (0, 0)),
    ],
    out_specs=pl.BlockSpec((_BT // 4, 128), lambda i: (i, 0)),
    out_shape=jax.ShapeDtypeStruct((E // 4, 128), jnp.float32),
    compiler_params=pltpu.CompilerParams(dimension_semantics=("parallel",)),
)


@functools.partial(
    pl.kernel,
    out_type=jax.ShapeDtypeStruct((NC, N, PAD), jnp.float32),
    mesh=_mesh,
    compiler_params=_sc_params,
    scratch_types=[
        pltpu.VMEM((NCH, CHUNK), jnp.int32),
        pltpu.VMEM((KFIRE, CHUNK, PAD), jnp.float32),
        pltpu.VMEM_SHARED((N, PAD), jnp.float32),
        pltpu.SemaphoreType.DMA,
    ],
)
def _scatter_tp(tp2, src2, zeros_hbm, out, idx_v, rows_v, acc, sem):
    c = lax.axis_index("c")
    s = lax.axis_index("s")
    w = s * NC + c
    pltpu.sync_copy(zeros_hbm.at[pl.ds(s * NPT, NPT), :],
                    acc.at[pl.ds(s * NPT, NPT), :])
    pltpu.sync_copy(src2.at[pl.ds(w * NCH, NCH), :], idx_v)
    plsc.subcore_barrier()

    def group(g, carry):
        base = g * KFIRE
        cps = [
            pltpu.async_copy(
                tp2.at[pl.ds((w * NCH + base + b) * CHUNK, CHUNK), :],
                rows_v.at[b], sem)
            for b in range(KFIRE)
        ]
        for cp in cps:
            cp.wait()
        for b in range(KFIRE):
            pltpu.sync_copy(rows_v.at[b], acc.at[idx_v.at[base + b]], add=True)
        return carry

    lax.fori_loop(0, NGRP, group, 0)
    plsc.subcore_barrier()
    pltpu.sync_copy(acc.at[pl.ds(s * NPT, NPT), :],
                    out.at[c, pl.ds(s * NPT, NPT), :])


def _np_c(f):
    a = np.zeros(f[0], np.float32)
    f[1](a)
    return a


def _fill_cnt(a):
    for j in range(4):
        a[32 * j + NM, 32 * j:32 * j + NM] = 1.0


def _fill_lo(a):
    for q in range(4):
        for f in range(NM):
            a[NM * q + f, 32 * q + f] = 1.0


def _fill_hi(a):
    for q in range(4):
        for f in range(NM):
            a[64 + NM * q + f, 32 * q + f] = 1.0


def _fill_g(a):
    for j in range(4):
        for f in range(NM):
            a[32 * j + f, f] = 1.0


def _fill_b(a):
    for j in range(4):
        for f in range(NM):
            a[f, 32 * j + f] = 1.0


def _fill_k(a):
    for q in range(4):
        for f in range(NM):
            a[32 * q + f, NM * q + f] = 1.0


def _fill_m(a):
    for j in range(4):
        a[0, 32 * j:32 * j + NM] = 1.0


_CNT = _np_c(((128, 128), _fill_cnt))
_ELO = _np_c(((128, 128), _fill_lo))
_EHI = _np_c(((128, 128), _fill_hi))
_G16 = _np_c(((128, NM), _fill_g))
_B16 = _np_c(((NM, 128), _fill_b))
_K64 = _np_c(((128, 64), _fill_k))
_MSK = _np_c(((1, 128), _fill_m))


def _combine_bn_body(p, nf2, gamr, betr, cntm, elo, ehi, g16, b16, k64, mskm,
                     out):
    t4 = p[0] + p[1]
    cntb = t4 @ cntm[...]
    msk = mskm[...]
    nf4 = jnp.stack(
        [nf2[...] @ elo[...], nf2[...] @ ehi[...]],
        axis=1).reshape(N // 4, 128)
    o4 = t4 * msk / jnp.maximum(cntb, 1.0) + nf4
    mu = (jnp.sum(o4, axis=0, keepdims=True) @ g16[...]) / N
    mu4 = mu @ b16[...]
    d = (o4 - mu4) * msk
    var = (jnp.sum(d * d, axis=0, keepdims=True) @ g16[...]) / N
    scale4 = (lax.rsqrt(var + 1e-5) * gamr[...]) @ b16[...]
    b4 = betr[...] @ b16[...]
    o4 = d * scale4 + b4
    o2 = o4.reshape(N // 8, 2, 128)
    out[...] = jnp.concatenate(
        [o2[:, 0, :] @ k64[...], o2[:, 1, :] @ k64[...]], axis=1)


_combine_bn = pl.pallas_call(
    _combine_bn_body,
    out_shape=jax.ShapeDtypeStruct((N // 8, 128), jnp.float32),
)


@functools.partial(
    pl.kernel,
    out_type=(jax.ShapeDtypeStruct((E, NM), jnp.float32),
              jax.ShapeDtypeStruct((E, NM), jnp.float32)),
    mesh=_mesh,
    compiler_params=_sc_params,
    scratch_types=[
        pltpu.VMEM((NCH, CHUNK), jnp.int32),
        pltpu.VMEM((NCH, CHUNK), jnp.int32),
        pltpu.VMEM((EPW, NM), jnp.float32),
        pltpu.SemaphoreType.DMA,
    ],
)
def _gather_out(table, dst2, src2, o_dst, o_src, idx_d, idx_s, rows_v, sem):
    w = lax.axis_index("s") * NC + lax.axis_index("c")
    pltpu.sync_copy(dst2.at[pl.ds(w * NCH, NCH), :], idx_d)
    pltpu.sync_copy(src2.at[pl.ds(w * NCH, NCH), :], idx_s)

    def run(idx_v, dst_hbm):
        def group(g, carry):
            base = g * KFIRE
            cps = [
                pltpu.async_copy(
                    table.at[idx_v.at[base + b]],
                    rows_v.at[pl.ds((base + b) * CHUNK, CHUNK), :],
                    sem,
                )
                for b in range(KFIRE)
            ]
            for cp in cps:
                cp.wait()
            return carry

        lax.fori_loop(0, NGRP, group, 0)
        pltpu.sync_copy(rows_v, dst_hbm.at[pl.ds(w * EPW, EPW), :])

    run(idx_d, o_dst)
    run(idx_s, o_src)


def _edge_update_body(eft, od_p, os_p, lin, w1, b1c, w2t, b2c, w3t, b3c,
                      lngc, lnbc, out):
    eye = jnp.eye(NM, dtype=jnp.float32)
    odt = _unpack16(od_p[...], eye)
    ost = _unpack16(os_p[...], eye)
    lin4 = lin[...] * 0.25
    ad = (lin4 @ w1[0:H, :]).T
    as_ = (lin4 @ w1[H:2 * H, :]).T
    mt = (ad @ odt + as_ @ ost + w1[2 * H:, :].T @ eft[...]
          + b1c[...])
    mt = jnp.maximum(mt, 0.0)
    mt = jnp.maximum(w2t[...] @ mt + b2c[...], 0.0)
    mt = w3t[...] @ mt + b3c[...]
    ef2 = eft[...] + mt
    mu = jnp.mean(ef2, axis=0, keepdims=True)
    var = jnp.mean((ef2 - mu) ** 2, axis=0, keepdims=True)
    out[...] = (ef2 - mu) * lax.rsqrt(var + 1e-5) * lngc[...] + lnbc[...]


_edge_update = pl.pallas_call(
    _edge_update_body,
    grid=(E // _BT,),
    in_specs=[
        pl.BlockSpec((H, _BT), lambda i: (0, i)),
        pl.BlockSpec((_BT // 8, 128), lambda i: (i, 0)),
        pl.BlockSpec((_BT // 8, 128), lambda i: (i, 0)),
        pl.BlockSpec((NM, H), lambda i: (0, 0)),
        pl.BlockSpec((3 * H, H), lambda i: (0, 0)),
        pl.BlockSpec((H, 1), lambda i: (0, 0)),
        pl.BlockSpec((H, H), lambda i: (0, 0)),
        pl.BlockSpec((H, 1), lambda i: (0, 0)),
        pl.BlockSpec((H, H), lambda i: (0, 0)),
        pl.BlockSpec((H, 1), lambda i: (0, 0)),
        pl.BlockSpec((H, 1), lambda i: (0, 0)),
        pl.BlockSpec((H, 1), lambda i: (0, 0)),
    ],
    out_specs=pl.BlockSpec((H, _BT), lambda i: (0, i)),
    out_shape=jax.ShapeDtypeStruct((H, E), jnp.float32),
    compiler_params=pltpu.CompilerParams(dimension_semantics=("parallel",)),
)


def kernel(node_features, edge_features, edge_sh, edge_index,
           W_fc1, b_fc1, W_fc2, b_fc2, bn_gamma, bn_beta,
           eu_lin_W, eu_W1, eu_b1, eu_W2, eu_b2, eu_W3, eu_b3,
           ln_gamma, ln_beta):
    dstp = (edge_index[0].reshape(E // _BT, 8, _BT // 8)
            .swapaxes(1, 2).reshape(E // CHUNK, CHUNK))
    srcp = (edge_index[1].reshape(E // _BT, 8, _BT // 8)
            .swapaxes(1, 2).reshape(E // CHUNK, CHUNK))
    srcp4 = (edge_index[1].reshape(E // _BT, 4, _BT // 4)
             .swapaxes(1, 2).reshape(E // CHUNK, CHUNK))
    eft = edge_features.T
    sht = edge_sh.T

    x = _gather_x(node_features, dstp)
    tp = _edge_tp(eft, x.reshape(E // 8, 128), sht,
                  W_fc1.T, b_fc1.reshape(H, 1),
                  W_fc2.T, b_fc2.reshape(WN, 1))
    partials = _scatter_tp(tp.reshape(E, PAD), srcp4,
                           jnp.zeros((N, PAD), jnp.float32))
    out_sc = _combine_bn(partials.reshape(NC, N // 4, 128),
                         node_features.reshape(N // 8, 128),
                         bn_gamma.reshape(1, NM),
                         bn_beta.reshape(1, NM),
                         jnp.asarray(_CNT), jnp.asarray(_ELO),
                         jnp.asarray(_EHI), jnp.asarray(_G16),
                         jnp.asarray(_B16), jnp.asarray(_K64),
                         jnp.asarray(_MSK)).reshape(N, NM)
    o_dst, o_src = _gather_out(out_sc, dstp, srcp)
    eft_out = _edge_update(eft, o_dst.reshape(E // 8, 128),
                           o_src.reshape(E // 8, 128),
                           eu_lin_W, eu_W1,
                           eu_b1.reshape(H, 1), eu_W2.T,
                           eu_b2.reshape(H, 1), eu_W3.T,
                           eu_b3.reshape(H, 1),
                           ln_gamma.reshape(H, 1),
                           ln_beta.reshape(H, 1))
    return (out_sc, eft_out.T)
